# baseline (speedup 1.0000x reference)
import functools

import jax
import jax.numpy as jnp
from jax import lax
from jax.experimental import pallas as pl
from jax.experimental.pallas import tpu as pltpu
from jax.experimental.pallas import tpu_sc as plsc


def _build_sc_gather(N, DP, n_per_w, chunk, NC):
    n_chunks = n_per_w // chunk
    n_pairs = n_chunks // 2
    mesh = plsc.VectorSubcoreMesh(core_axis_name="c", subcore_axis_name="s")

    @functools.partial(
        pl.kernel,
        mesh=mesh,
        out_type=jax.ShapeDtypeStruct((N, DP), jnp.float32),
        scratch_types=[
            pltpu.VMEM((n_per_w,), jnp.int32),
            pltpu.VMEM((chunk, DP), jnp.float32),
            pltpu.VMEM((chunk, DP), jnp.float32),
            pltpu.SemaphoreType.DMA,
            pltpu.SemaphoreType.DMA,
            pltpu.SemaphoreType.DMA,
            pltpu.SemaphoreType.DMA,
        ],
        compiler_params=pltpu.CompilerParams(use_tc_tiling_on_sc=True),
    )
    def k(idx_hbm, table_hbm, out_hbm, idx_v, buf0, buf1, sg0, sg1, ss0, ss1):
        wid = lax.axis_index("s") * NC + lax.axis_index("c")
        base = wid * n_per_w
        pltpu.sync_copy(idx_hbm.at[pl.ds(base, n_per_w)], idx_v)

        def gather(c, buf, sem):
            return pltpu.async_copy(
                table_hbm.at[idx_v.at[pl.ds(c * chunk, chunk)]], buf, sem
            )

        def store(c, buf, sem):
            return pltpu.async_copy(buf, out_hbm.at[pl.ds(base + c * chunk, chunk)], sem)

        def wait_gather(buf, sem):
            pltpu.make_async_copy(
                table_hbm.at[idx_v.at[pl.ds(0, chunk)]], buf, sem
            ).wait()

        gather(0, buf0, sg0)

        def body(p, carry):
            c0 = 2 * p
            c1 = c0 + 1
            g1 = gather(c1, buf1, sg1)
            wait_gather(buf0, sg0)
            s0 = store(c0, buf0, ss0)
            g1.wait()
            s1 = store(c1, buf1, ss1)
            s0.wait()
            gather(lax.min(c0 + 2, n_chunks - 1), buf0, sg0)
            s1.wait()
            return carry

        lax.fori_loop(0, n_pairs, body, 0)
        wait_gather(buf0, sg0)

    return k


def kernel(inputs, table):
    B, F = inputs.shape
    V, D = table.shape
    N = B * F
    DP = 2 * D
    flat_idx = inputs.reshape(N).astype(jnp.int32)
    one = (flat_idx[0] * 0 + 1).astype(jnp.float32)
    tpad = jnp.pad(table, ((0, 0), (0, DP - D))) * one

    info = plsc.get_sparse_core_info()
    NC, NS = info.num_cores, info.num_subcores
    NW = NC * NS
    n_per_w = N // NW
    chunk = 416

    k = _build_sc_gather(N, DP, n_per_w, chunk, NC)
    out = k(flat_idx, tpad)
    return out[:, :D].reshape(B, F, D)


# speedup vs baseline: 1.0567x; 1.0567x over previous
import functools

import jax
import jax.numpy as jnp
from jax import lax
from jax.experimental import pallas as pl
from jax.experimental.pallas import tpu as pltpu
from jax.experimental.pallas import tpu_sc as plsc


def _build(N, V, D, n_per_w, chunk, NC):
    n_chunks = n_per_w // chunk
    n_pairs = n_chunks // 2
    mesh = plsc.VectorSubcoreMesh(core_axis_name="c", subcore_axis_name="s")

    @functools.partial(
        pl.kernel,
        mesh=mesh,
        out_type=jax.ShapeDtypeStruct((N, D), jnp.float32),
        scratch_types=[
            pltpu.VMEM((n_per_w,), jnp.int32),
            pltpu.VMEM((chunk, 2 * D), jnp.float32),
            pltpu.VMEM((chunk, 2 * D), jnp.float32),
            pltpu.SemaphoreType.DMA,
            pltpu.SemaphoreType.DMA,
            pltpu.SemaphoreType.DMA,
            pltpu.SemaphoreType.DMA,
        ],
        compiler_params=pltpu.CompilerParams(use_tc_tiling_on_sc=False),
    )
    def k(idx_hbm, table_hbm, out_hbm, idx_v, buf0, buf1, sg0, sg1, ss0, ss1):
        tref = table_hbm
        wid = lax.axis_index("s") * NC + lax.axis_index("c")
        base = wid * n_per_w
        pltpu.sync_copy(idx_hbm.at[pl.ds(base, n_per_w)], idx_v)

        def gather(c, buf, sem):
            pltpu.async_copy(tref.at[idx_v.at[pl.ds(c * chunk, chunk)]], buf, sem)

        def wait_gather(buf, sem):
            pltpu.make_async_copy(
                tref.at[idx_v.at[pl.ds(0, chunk)]], buf, sem
            ).wait()

        def store(c, buf, sem):
            pltpu.async_copy(
                buf.at[:, pl.ds(0, D)],
                out_hbm.at[pl.ds(base + c * chunk, chunk)],
                sem,
            )

        def wait_store(buf, sem):
            pltpu.make_async_copy(
                buf.at[:, pl.ds(0, D)],
                out_hbm.at[pl.ds(base, chunk)],
                sem,
            ).wait()

        gather(0, buf0, sg0)

        def body(p, carry):
            c0 = 2 * p
            c1 = c0 + 1
            gather(c1, buf1, sg1)
            wait_gather(buf0, sg0)
            store(c0, buf0, ss0)
            wait_gather(buf1, sg1)
            store(c1, buf1, ss1)
            wait_store(buf0, ss0)
            gather(lax.min(c0 + 2, n_chunks - 1), buf0, sg0)
            wait_store(buf1, ss1)
            return carry

        lax.fori_loop(0, n_pairs, body, 0)
        wait_gather(buf0, sg0)

    return k


def kernel(inputs, table):
    B, F = inputs.shape
    V, D = table.shape
    N = B * F
    flat_idx = inputs.reshape(N).astype(jnp.int32)

    one = (flat_idx[0] * 0 + 1).astype(jnp.float32)
    tpad = (
        jnp.pad(table.reshape(V // 8, 8, D), ((0, 0), (0, 0), (0, D))) * one
    ).reshape(V, 2 * D)

    info = plsc.get_sparse_core_info()
    NC, NS = info.num_cores, info.num_subcores
    NW = NC * NS
    n_per_w = N // NW
    chunk = 416

    k = _build(N, V, D, n_per_w, chunk, NC)
    out = k(flat_idx, tpad)
    return out.reshape(B, F, D)


# speedup vs baseline: 1.1534x; 1.0915x over previous
import functools

import jax
import jax.numpy as jnp
from jax import lax
from jax.experimental import pallas as pl
from jax.experimental.pallas import tpu as pltpu
from jax.experimental.pallas import tpu_sc as plsc


def _build(B, F, D, V, n_per_w, chunk, NC):
    n_chunks = n_per_w // chunk
    n_pairs = n_chunks // 2
    bpc = chunk // F
    mesh = plsc.VectorSubcoreMesh(core_axis_name="c", subcore_axis_name="s")

    @functools.partial(
        pl.kernel,
        mesh=mesh,
        out_type=jax.ShapeDtypeStruct((B, F, D), jnp.float32),
        scratch_types=[
            pltpu.VMEM((n_per_w,), jnp.int32),
            pltpu.VMEM((chunk, D), jnp.float32),
            pltpu.VMEM((chunk, D), jnp.float32),
            pltpu.SemaphoreType.DMA,
            pltpu.SemaphoreType.DMA,
            pltpu.SemaphoreType.DMA,
            pltpu.SemaphoreType.DMA,
        ],
        compiler_params=pltpu.CompilerParams(use_tc_tiling_on_sc=False),
    )
    def k(idx_hbm, table_hbm, out_hbm, idx_v, buf0, buf1, sg0, sg1, ss0, ss1):
        wid = lax.axis_index("s") * NC + lax.axis_index("c")
        base = wid * n_per_w
        b_base = wid * (n_per_w // F)
        pltpu.sync_copy(idx_hbm.at[pl.ds(base, n_per_w)], idx_v)

        def gather(c, buf, sem):
            pltpu.async_copy(
                table_hbm.at[idx_v.at[pl.ds(c * chunk, chunk)]], buf, sem
            )

        def wait_gather(buf, sem):
            pltpu.make_async_copy(
                table_hbm.at[idx_v.at[pl.ds(0, chunk)]], buf, sem
            ).wait()

        def store(c, buf, sem):
            b0 = b_base + c * bpc
            for q in range(bpc):
                pltpu.async_copy(
                    buf.at[pl.ds(q * F, F)], out_hbm.at[b0 + q], sem
                )

        def wait_store(sem):
            pltpu.make_async_copy(
                out_hbm.at[pl.ds(0, bpc)], out_hbm.at[pl.ds(bpc, bpc)], sem
            ).wait()

        gather(0, buf0, sg0)

        def body(p, carry):
            c0 = 2 * p
            c1 = c0 + 1
            gather(c1, buf1, sg1)
            wait_gather(buf0, sg0)
            store(c0, buf0, ss0)
            wait_gather(buf1, sg1)
            store(c1, buf1, ss1)
            wait_store(ss0)
            gather(lax.min(c0 + 2, n_chunks - 1), buf0, sg0)
            wait_store(ss1)
            return carry

        lax.fori_loop(0, n_pairs, body, 0)
        wait_gather(buf0, sg0)

    return k


def kernel(inputs, table):
    B, F = inputs.shape
    V, D = table.shape
    N = B * F
    flat_idx = inputs.reshape(N).astype(jnp.int32) * 2

    one = (flat_idx[0] * 0 + 1).astype(jnp.float32)
    tpad = (
        jnp.pad(table.reshape(V // 8, 8, D), ((0, 0), (0, 0), (0, D))) * one
    ).reshape(2 * V, D)

    info = plsc.get_sparse_core_info()
    NC, NS = info.num_cores, info.num_subcores
    NW = NC * NS
    n_per_w = N // NW
    chunk = 416

    k = _build(B, F, D, V, n_per_w, chunk, NC)
    return k(flat_idx, tpad)
